# baseline (speedup 1.0000x reference)
import functools

import jax
import jax.numpy as jnp
from jax import lax
from jax.experimental import pallas as pl
from jax.experimental.pallas import tpu as pltpu
from jax.experimental.pallas import tpu_sc as plsc

TOTAL = 32768
B = 16
C_IN = 32
C_OUT = 128
L = 16
NS = 16
NCHUNK = 4
CH = 514
RPW = NCHUNK * CH
PSZ = B * C_IN
NGRP = 8
GW = C_OUT // NGRP


def _round_bf16(v):
    i = plsc.bitcast(v, jnp.int32)
    i = (i + jnp.int32(0x7FFF) + ((i >> 16) & jnp.int32(1))) & jnp.int32(-65536)
    return plsc.bitcast(i, jnp.float32)


def _sc_body(x_hbm, len_hbm, w_hbm, b_hbm, g_hbm, bt_hbm, out_hbm,
             len_vm, buf, part, comb, pooled_vm, w_vm, bgb_vm, y_vm,
             shared, sem0, sem1, sem2, sem3, osem):
    c = lax.axis_index("c")
    s = lax.axis_index("s")

    pltpu.sync_copy(len_hbm, len_vm)

    lenv = len_vm[...]
    csum = []
    acc = jnp.int32(0)
    for j in range(B):
        acc = acc + lenv[j]
        csum.append(acc)
    total = csum[-1]

    lo = (s * total) // NS
    hi = ((s + 1) * total) // NS
    base = (jnp.minimum(lo, TOTAL - RPW) // 8) * 8

    sems = [sem0, sem1, sem2, sem3]
    copies = [
        pltpu.async_copy(
            x_hbm.at[pl.ds((base + t * CH) * C_IN, CH * C_IN)],
            buf.at[pl.ds(t * CH * C_IN, CH * C_IN)],
            sems[t],
        )
        for t in range(NCHUNK)
    ]

    neg = jnp.full((L,), -jnp.inf, jnp.float32)
    for j in range(2 * B):
        part[pl.ds(j * L, L)] = neg

    for t in range(NCHUNK):
        copies[t].wait()
        cb = base + t * CH
        prev = jnp.int32(0)
        for j in range(B):
            r0 = jnp.maximum(jnp.maximum(prev, lo), cb)
            r1 = jnp.minimum(jnp.minimum(csum[j], hi), cb + CH)
            prev = csum[j]
            r1 = jnp.maximum(r1, r0)
            p0 = r0 - base
            p1 = r1 - base
            a0 = part[pl.ds(j * C_IN, L)]
            a1 = part[pl.ds(j * C_IN + L, L)]

            n8 = (p1 - p0) >> 3

            def body8(i, carry, p0=p0):
                b0, b1 = carry
                q = (p0 + i * 8) * C_IN
                for u in range(8):
                    b0 = jnp.maximum(b0, buf[pl.ds(q + u * C_IN, L)])
                    b1 = jnp.maximum(b1, buf[pl.ds(q + u * C_IN + L, L)])
                return (b0, b1)

            def body1(p, carry):
                b0, b1 = carry
                q = p * C_IN
                b0 = jnp.maximum(b0, buf[pl.ds(q, L)])
                b1 = jnp.maximum(b1, buf[pl.ds(q + L, L)])
                return (b0, b1)

            a0, a1 = lax.fori_loop(0, n8, body8, (a0, a1))
            a0, a1 = lax.fori_loop(p0 + n8 * 8, p1, body1, (a0, a1))
            part[pl.ds(j * C_IN, L)] = a0
            part[pl.ds(j * C_IN + L, L)] = a1

    pltpu.sync_copy(part, shared.at[pl.ds(s * PSZ, PSZ)])
    plsc.subcore_barrier()

    @pl.when(s < (NGRP // 2))
    def _epilogue():
        g = c * (NGRP // 2) + s
        pltpu.sync_copy(shared, comb)
        pltpu.sync_copy(w_hbm, w_vm)
        pltpu.sync_copy(b_hbm.at[pl.ds(g * GW, GW)], bgb_vm.at[pl.ds(0, GW)])
        pltpu.sync_copy(g_hbm.at[pl.ds(g * GW, GW)], bgb_vm.at[pl.ds(GW, GW)])
        pltpu.sync_copy(bt_hbm.at[pl.ds(g * GW, GW)],
                        bgb_vm.at[pl.ds(2 * GW, GW)])

        for j in range(B):
            for h in range(2):
                m = comb[pl.ds(j * C_IN + h * L, L)]
                for w in range(1, NS):
                    m = jnp.maximum(m, comb[pl.ds(w * PSZ + j * C_IN + h * L, L)])
                pooled_vm[pl.ds(j * C_IN + h * L, L)] = _round_bf16(m)

        lanes = lax.iota(jnp.int32, L) * C_IN + g * (GW * C_IN)
        wt = []
        for k in range(C_IN):
            wt.append(_round_bf16(plsc.load_gather(w_vm, [lanes + k])))

        bias = bgb_vm[pl.ds(0, GW)]
        for b0 in range(0, B, 4):
            ys = [jnp.zeros((L,), jnp.float32) for _ in range(4)]
            pv = [pooled_vm[pl.ds((b0 + u) * C_IN + h * L, L)]
                  for u in range(4) for h in range(2)]
            for k in range(C_IN):
                for u in range(4):
                    ys[u] = ys[u] + pv[2 * u + k // L][k % L] * wt[k]
            for u in range(4):
                y_vm[pl.ds((b0 + u) * GW, GW)] = ys[u] + bias

        mean = y_vm[pl.ds(0, GW)]
        for b in range(1, B):
            mean = mean + y_vm[pl.ds(b * GW, GW)]
        mean = mean * (1.0 / B)
        var = jnp.zeros((L,), jnp.float32)
        for b in range(B):
            d = y_vm[pl.ds(b * GW, GW)] - mean
            var = var + d * d
        var = var * (1.0 / B)

        t = var + 1e-5
        i = plsc.bitcast(t, jnp.int32)
        i = jnp.int32(0x5F3759DF) - (i >> 1)
        r = plsc.bitcast(i, jnp.float32)
        for _ in range(3):
            r = r * (1.5 - 0.5 * t * r * r)

        gam = bgb_vm[pl.ds(GW, GW)]
        bet = bgb_vm[pl.ds(2 * GW, GW)]
        for b in range(B):
            yn = ((y_vm[pl.ds(b * GW, GW)] - mean) * r) * gam + bet
            y_vm[pl.ds(b * GW, GW)] = jnp.maximum(yn, 0.0)

        outs = [
            pltpu.async_copy(
                y_vm.at[pl.ds(b * GW, GW)],
                out_hbm.at[pl.ds(b * C_OUT + g * GW, GW)],
                osem,
            )
            for b in range(B)
        ]
        for o in outs:
            o.wait()


_sc_call = functools.partial(
    pl.kernel,
    out_type=jax.ShapeDtypeStruct((B * C_OUT,), jnp.float32),
    mesh=plsc.VectorSubcoreMesh(core_axis_name="c", subcore_axis_name="s"),
    compiler_params=pltpu.CompilerParams(needs_layout_passes=False),
    scratch_types=[
        pltpu.VMEM((B,), jnp.int32),
        pltpu.VMEM((RPW * C_IN,), jnp.float32),
        pltpu.VMEM((PSZ,), jnp.float32),
        pltpu.VMEM((NS * PSZ,), jnp.float32),
        pltpu.VMEM((PSZ,), jnp.float32),
        pltpu.VMEM((C_OUT * C_IN,), jnp.float32),
        pltpu.VMEM((3 * GW,), jnp.float32),
        pltpu.VMEM((B * GW,), jnp.float32),
        pltpu.VMEM_SHARED((NS * PSZ,), jnp.float32),
        pltpu.SemaphoreType.DMA,
        pltpu.SemaphoreType.DMA,
        pltpu.SemaphoreType.DMA,
        pltpu.SemaphoreType.DMA,
        pltpu.SemaphoreType.DMA,
    ],
)(_sc_body)


def kernel(x, length, W, b, gamma, beta):
    out = _sc_call(x.reshape(-1), length.astype(jnp.int32), W.reshape(-1),
                   b, gamma, beta)
    return out.reshape(B, C_OUT)


# speedup vs baseline: 1.2803x; 1.2803x over previous
import functools

import jax
import jax.numpy as jnp
from jax import lax
from jax.experimental import pallas as pl
from jax.experimental.pallas import tpu as pltpu
from jax.experimental.pallas import tpu_sc as plsc

TOTAL = 32768
B = 16
C_IN = 32
C_OUT = 128
NW = 32
NCHUNK = 2
CH = 520
RPW = NCHUNK * CH
L = 16
PSZ = B * C_IN


def _sc_partial_max(x_hbm, len_hbm, out_hbm, len_vm, buf, part, sem0, sem1):
    c = lax.axis_index("c")
    s = lax.axis_index("s")
    w = s * 2 + c

    pltpu.sync_copy(len_hbm, len_vm)

    lenv = len_vm[...]
    csum = []
    acc = jnp.int32(0)
    for j in range(B):
        acc = acc + lenv[j]
        csum.append(acc)
    total = csum[-1]

    lo = (w * total) // NW
    hi = ((w + 1) * total) // NW
    base = (jnp.minimum(lo, TOTAL - RPW) // 8) * 8

    sems = [sem0, sem1]
    for t in range(NCHUNK):
        @pl.when(base + t * CH < hi)
        def _fire(t=t):
            pltpu.async_copy(
                x_hbm.at[pl.ds((base + t * CH) * C_IN, CH * C_IN)],
                buf.at[pl.ds(t * CH * C_IN, CH * C_IN)],
                sems[t],
            )

    neg = jnp.full((L,), -jnp.inf, jnp.float32)
    for j in range(2 * B):
        part[pl.ds(j * L, L)] = neg

    for t in range(NCHUNK):
        @pl.when(base + t * CH < hi)
        def _consume(t=t):
            pltpu.make_async_copy(
                x_hbm.at[pl.ds((base + t * CH) * C_IN, CH * C_IN)],
                buf.at[pl.ds(t * CH * C_IN, CH * C_IN)],
                sems[t],
            ).wait()
            cb = base + t * CH
            prev = jnp.int32(0)
            for j in range(B):
                r0 = jnp.maximum(jnp.maximum(prev, lo), cb)
                r1 = jnp.minimum(jnp.minimum(csum[j], hi), cb + CH)
                prev = csum[j]
                r1 = jnp.maximum(r1, r0)
                p0 = r0 - base
                p1 = r1 - base
                a0 = part[pl.ds(j * C_IN, L)]
                a1 = part[pl.ds(j * C_IN + L, L)]

                n8 = (p1 - p0) >> 3

                def body8(i, carry, p0=p0):
                    b0, b1 = carry
                    q = (p0 + i * 8) * C_IN
                    for u in range(8):
                        b0 = jnp.maximum(b0, buf[pl.ds(q + u * C_IN, L)])
                        b1 = jnp.maximum(b1, buf[pl.ds(q + u * C_IN + L, L)])
                    return (b0, b1)

                def body1(p, carry):
                    b0, b1 = carry
                    q = p * C_IN
                    b0 = jnp.maximum(b0, buf[pl.ds(q, L)])
                    b1 = jnp.maximum(b1, buf[pl.ds(q + L, L)])
                    return (b0, b1)

                a0, a1 = lax.fori_loop(0, n8, body8, (a0, a1))
                a0, a1 = lax.fori_loop(p0 + n8 * 8, p1, body1, (a0, a1))
                part[pl.ds(j * C_IN, L)] = a0
                part[pl.ds(j * C_IN + L, L)] = a1

    pltpu.sync_copy(part, out_hbm.at[pl.ds(w * PSZ, PSZ)])


_sc_call = functools.partial(
    pl.kernel,
    out_type=jax.ShapeDtypeStruct((NW * PSZ,), jnp.float32),
    mesh=plsc.VectorSubcoreMesh(core_axis_name="c", subcore_axis_name="s"),
    scratch_types=[
        pltpu.VMEM((B,), jnp.int32),
        pltpu.VMEM((RPW * C_IN,), jnp.float32),
        pltpu.VMEM((PSZ,), jnp.float32),
        pltpu.SemaphoreType.DMA,
        pltpu.SemaphoreType.DMA,
    ],
)(_sc_partial_max)


def _tc_body(part_ref, w_ref, b_ref, g_ref, bt_ref, o_ref):
    pooled = jnp.max(part_ref[...], axis=0)
    y = lax.dot_general(
        pooled, w_ref[...], (((1,), (1,)), ((), ())),
        preferred_element_type=jnp.float32,
    )
    y = y + b_ref[...]
    mean = jnp.mean(y, axis=0, keepdims=True)
    var = jnp.mean(jnp.square(y - mean), axis=0, keepdims=True)
    yn = (y - mean) / jnp.sqrt(var + 1e-5) * g_ref[...] + bt_ref[...]
    o_ref[...] = jnp.maximum(yn, 0.0)


_tc_call = pl.pallas_call(
    _tc_body,
    out_shape=jax.ShapeDtypeStruct((B, C_OUT), jnp.float32),
)


def kernel(x, length, W, b, gamma, beta):
    part = _sc_call(x.reshape(-1), length.astype(jnp.int32))
    part = part.reshape(NW, B, C_IN)
    return _tc_call(part, W, b.reshape(1, C_OUT), gamma.reshape(1, C_OUT),
                    beta.reshape(1, C_OUT))


# speedup vs baseline: 1.2839x; 1.0028x over previous
import functools

import jax
import jax.numpy as jnp
from jax import lax
from jax.experimental import pallas as pl
from jax.experimental.pallas import tpu as pltpu
from jax.experimental.pallas import tpu_sc as plsc

TOTAL = 32768
B = 16
C_IN = 32
C_OUT = 128
NW = 32
NCHUNK = 2
CH = 520
RPW = NCHUNK * CH
L = 16
PSZ = B * C_IN


def _sc_partial_max(x_hbm, len_hbm, out_hbm, len_vm, buf, part, sem0, sem1):
    c = lax.axis_index("c")
    s = lax.axis_index("s")
    w = s * 2 + c

    pltpu.sync_copy(len_hbm, len_vm)

    lenv = len_vm[...]
    csum = []
    acc = jnp.int32(0)
    for j in range(B):
        acc = acc + lenv[j]
        csum.append(acc)
    total = csum[-1]

    lo = (w * total) // NW
    hi = ((w + 1) * total) // NW
    base = (jnp.minimum(lo, TOTAL - RPW) // 8) * 8

    sems = [sem0, sem1]
    for t in range(NCHUNK):
        @pl.when(base + t * CH < hi)
        def _fire(t=t):
            pltpu.async_copy(
                x_hbm.at[pl.ds((base + t * CH) * C_IN, CH * C_IN)],
                buf.at[pl.ds(t * CH * C_IN, CH * C_IN)],
                sems[t],
            )

    neg = jnp.full((L,), -jnp.inf, jnp.float32)
    for j in range(2 * B):
        part[pl.ds(j * L, L)] = neg

    for t in range(NCHUNK):
        @pl.when(base + t * CH < hi)
        def _consume(t=t):
            pltpu.make_async_copy(
                x_hbm.at[pl.ds((base + t * CH) * C_IN, CH * C_IN)],
                buf.at[pl.ds(t * CH * C_IN, CH * C_IN)],
                sems[t],
            ).wait()
            cb = base + t * CH
            prev = jnp.int32(0)
            for j in range(B):
                r0 = jnp.maximum(jnp.maximum(prev, lo), cb)
                r1 = jnp.minimum(jnp.minimum(csum[j], hi), cb + CH)
                prev = csum[j]
                r1 = jnp.maximum(r1, r0)
                p0 = r0 - base
                p1 = r1 - base
                a0 = part[pl.ds(j * C_IN, L)]
                a1 = part[pl.ds(j * C_IN + L, L)]

                n8 = (p1 - p0) >> 3

                def body8(i, carry, p0=p0):
                    b0, b1 = carry
                    q = (p0 + i * 8) * C_IN
                    for u in range(8):
                        b0 = jnp.maximum(b0, buf[pl.ds(q + u * C_IN, L)])
                        b1 = jnp.maximum(b1, buf[pl.ds(q + u * C_IN + L, L)])
                    return (b0, b1)

                def body1(p, carry):
                    b0, b1 = carry
                    q = p * C_IN
                    b0 = jnp.maximum(b0, buf[pl.ds(q, L)])
                    b1 = jnp.maximum(b1, buf[pl.ds(q + L, L)])
                    return (b0, b1)

                a0, a1 = lax.fori_loop(0, n8, body8, (a0, a1))
                a0, a1 = lax.fori_loop(p0 + n8 * 8, p1, body1, (a0, a1))
                part[pl.ds(j * C_IN, L)] = a0
                part[pl.ds(j * C_IN + L, L)] = a1

    pltpu.sync_copy(part, out_hbm.at[pl.ds(w * PSZ, PSZ)])


_sc_call = functools.partial(
    pl.kernel,
    out_type=jax.ShapeDtypeStruct((NW * PSZ,), jnp.float32),
    mesh=plsc.VectorSubcoreMesh(core_axis_name="c", subcore_axis_name="s"),
    compiler_params=pltpu.CompilerParams(
        skip_device_barrier=True,
        disable_bounds_checks=True,
        disable_semaphore_checks=True,
    ),
    scratch_types=[
        pltpu.VMEM((B,), jnp.int32),
        pltpu.VMEM((RPW * C_IN,), jnp.float32),
        pltpu.VMEM((PSZ,), jnp.float32),
        pltpu.SemaphoreType.DMA,
        pltpu.SemaphoreType.DMA,
    ],
)(_sc_partial_max)


def _tc_body(part_ref, w_ref, b_ref, g_ref, bt_ref, o_ref):
    pooled = jnp.max(part_ref[...], axis=0)
    y = lax.dot_general(
        pooled, w_ref[...], (((1,), (1,)), ((), ())),
        preferred_element_type=jnp.float32,
    )
    y = y + b_ref[...]
    mean = jnp.mean(y, axis=0, keepdims=True)
    var = jnp.mean(jnp.square(y - mean), axis=0, keepdims=True)
    yn = (y - mean) / jnp.sqrt(var + 1e-5) * g_ref[...] + bt_ref[...]
    o_ref[...] = jnp.maximum(yn, 0.0)


_tc_call = pl.pallas_call(
    _tc_body,
    out_shape=jax.ShapeDtypeStruct((B, C_OUT), jnp.float32),
)


def kernel(x, length, W, b, gamma, beta):
    part = _sc_call(x.reshape(-1), length.astype(jnp.int32))
    part = part.reshape(NW, B, C_IN)
    return _tc_call(part, W, b.reshape(1, C_OUT), gamma.reshape(1, C_OUT),
                    beta.reshape(1, C_OUT))
